# baseline (speedup 1.0000x reference)
import functools
import math

import jax
import jax.numpy as jnp
from jax import lax
from jax.experimental import pallas as pl
from jax.experimental.pallas import tpu as pltpu
from jax.experimental.pallas import tpu_sc as plsc

D_MODEL = 64
SCALE = math.sqrt(D_MODEL)
LANES = 16
VOCAB = 1000000
TILE_W = 128


def _pack_kernel():
    info = plsc.get_sparse_core_info()
    nw = info.num_cores * info.num_subcores
    n_blocks = VOCAB // TILE_W
    rem = VOCAB - n_blocks * TILE_W
    per_w = 245
    assert nw * per_w >= n_blocks

    mesh = plsc.VectorSubcoreMesh(core_axis_name="c", subcore_axis_name="s")

    @functools.partial(
        pl.kernel,
        mesh=mesh,
        out_type=jax.ShapeDtypeStruct((VOCAB // 2, 2 * D_MODEL), jnp.float32),
        scratch_types=[
            pltpu.VMEM((D_MODEL, TILE_W), jnp.float32),
            pltpu.VMEM((D_MODEL, TILE_W), jnp.float32),
            pltpu.VMEM((D_MODEL, TILE_W), jnp.float32),
            pltpu.SemaphoreType.DMA,
            pltpu.SemaphoreType.DMA,
        ],
        compiler_params=pltpu.CompilerParams(needs_layout_passes=False),
    )
    def pack(lutt_hbm, tail_hbm, out_hbm, b0_v, b1_v, s_v, sem0, sem1):
        wid = lax.axis_index("s") * info.num_cores + lax.axis_index("c")
        start = wid * per_w
        bufs = (b0_v, b1_v)
        sems = (sem0, sem1)
        iota = lax.iota(jnp.int32, LANES)

        def fire(tc, buf):
            @pl.when(tc < n_blocks)
            def _():
                for tj in range(D_MODEL // 8):
                    pltpu.async_copy(
                        lutt_hbm.at[pl.ds(8 * tj, 8), pl.ds(tc * TILE_W, TILE_W)],
                        bufs[buf].at[pl.ds(8 * tj, 8)],
                        sems[buf],
                    )

        def drain(tc, buf):
            @pl.when(tc < n_blocks)
            def _():
                for tj in range(D_MODEL // 8):
                    pltpu.make_async_copy(
                        lutt_hbm.at[pl.ds(8 * tj, 8), pl.ds(tc * TILE_W, TILE_W)],
                        bufs[buf].at[pl.ds(8 * tj, 8)],
                        sems[buf],
                    ).wait()

        fire(start, 0)

        def pair_body(g, _):
            for b in range(2):
                k = 2 * g + b
                tc = start + k
                p, q = b, 1 - b

                @pl.when(k < per_w)
                def _():
                    drain(tc, p)

                @pl.when(k + 1 < per_w)
                def _():
                    fire(tc + 1, q)

                @pl.when((k < per_w) & (tc < n_blocks))
                def _():
                    @plsc.parallel_loop(0, D_MODEL, step=1, unroll=2)
                    def _(c):
                        for h in range(2):
                            col = jnp.full((LANES,), 2 * c + h, jnp.int32)
                            for jg in range(D_MODEL // LANES):
                                v = plsc.load_gather(
                                    bufs[p], [jg * LANES + iota, col]
                                )
                                s_v[c, pl.ds(h * D_MODEL + jg * LANES, LANES)] = v

                    pltpu.sync_copy(
                        s_v, out_hbm.at[pl.ds(tc * D_MODEL, D_MODEL)]
                    )
            return 0

        lax.fori_loop(0, (per_w + 1) // 2, pair_body, 0)

        @pl.when(wid == nw - 1)
        def _():
            for tj in range(D_MODEL // 8):
                pltpu.async_copy(
                    tail_hbm.at[pl.ds(8 * tj, 8)],
                    b0_v.at[pl.ds(8 * tj, 8)],
                    sem0,
                )
            for tj in range(D_MODEL // 8):
                pltpu.make_async_copy(
                    tail_hbm.at[pl.ds(8 * tj, 8)],
                    b0_v.at[pl.ds(8 * tj, 8)],
                    sem0,
                ).wait()

            @plsc.parallel_loop(0, rem // 2, step=1, unroll=2)
            def _(c):
                for h in range(2):
                    col = jnp.full((LANES,), 2 * c + h, jnp.int32)
                    for jg in range(D_MODEL // LANES):
                        v = plsc.load_gather(b0_v, [jg * LANES + iota, col])
                        s_v[c, pl.ds(h * D_MODEL + jg * LANES, LANES)] = v

            pltpu.sync_copy(
                s_v.at[pl.ds(0, rem // 2)],
                out_hbm.at[pl.ds(n_blocks * (TILE_W // 2), rem // 2)],
            )

    return pack


def _lookup_kernel(n_s, n_b):
    info = plsc.get_sparse_core_info()
    nw = info.num_cores * info.num_subcores
    n_tb = n_b // TILE_W
    n_units = n_s * n_tb
    per_w = n_units // nw
    assert n_units % nw == 0 and per_w % 2 == 0
    tb_shift = n_tb.bit_length() - 1
    assert 1 << tb_shift == n_tb

    mesh = plsc.VectorSubcoreMesh(core_axis_name="c", subcore_axis_name="s")

    @functools.partial(
        pl.kernel,
        mesh=mesh,
        out_type=jax.ShapeDtypeStruct((n_s, D_MODEL, n_b), jnp.float32),
        scratch_types=[
            pltpu.VMEM((TILE_W,), jnp.int32),
            pltpu.VMEM((TILE_W,), jnp.int32),
            pltpu.VMEM((TILE_W,), jnp.int32),
            pltpu.VMEM((TILE_W,), jnp.int32),
            pltpu.VMEM((TILE_W, 2 * D_MODEL), jnp.float32),
            pltpu.VMEM((TILE_W, 2 * D_MODEL), jnp.float32),
            pltpu.VMEM((D_MODEL, TILE_W), jnp.float32),
            pltpu.SemaphoreType.DMA,
            pltpu.SemaphoreType.DMA,
            pltpu.SemaphoreType.DMA,
        ],
        compiler_params=pltpu.CompilerParams(needs_layout_passes=False),
    )
    def lookup(xt_hbm, lut2_hbm, out_hbm, i0_v, i1_v, g0_v, g1_v,
               r0_v, r1_v, o_v, sem0, sem1, osem):
        wid = lax.axis_index("s") * info.num_cores + lax.axis_index("c")
        start = wid * per_w
        idxs = (i0_v, i1_v)
        gidxs = (g0_v, g1_v)
        rows = (r0_v, r1_v)
        sems = (sem0, sem1)
        iota = lax.iota(jnp.int32, LANES)

        def fire(u, buf):
            s = u >> tb_shift
            tb = u & (n_tb - 1)
            pltpu.sync_copy(
                xt_hbm.at[s, pl.ds(tb * TILE_W, TILE_W)], idxs[buf]
            )
            for k in range(TILE_W // LANES):
                ids = idxs[buf][pl.ds(k * LANES, LANES)]
                gidxs[buf][pl.ds(k * LANES, LANES)] = ids >> 1
            pltpu.async_copy(lut2_hbm.at[gidxs[buf]], rows[buf], sems[buf])

        def drain(buf):
            pltpu.make_async_copy(
                lut2_hbm.at[gidxs[buf]], rows[buf], sems[buf]
            ).wait()

        fire(start, 0)

        def pair_body(g, _):
            for b in range(2):
                u = start + 2 * g + b
                p, q = b, 1 - b
                drain(p)

                @pl.when(2 * g + b + 1 < per_w)
                def _():
                    fire(u + 1, q)

                for bg in range(TILE_W // LANES):
                    ids = idxs[p][pl.ds(bg * LANES, LANES)]
                    half = (ids & 1) << 6
                    brow = bg * LANES + iota

                    @plsc.parallel_loop(0, D_MODEL, step=1, unroll=4)
                    def _(j):
                        v = plsc.load_gather(rows[p], [brow, half + j])
                        o_v[j, pl.ds(bg * LANES, LANES)] = v * SCALE

                s = u >> tb_shift
                tb = u & (n_tb - 1)
                for tj in range(D_MODEL // 8):
                    pltpu.async_copy(
                        o_v.at[pl.ds(8 * tj, 8)],
                        out_hbm.at[s, pl.ds(8 * tj, 8), pl.ds(tb * TILE_W, TILE_W)],
                        osem,
                    )
                for tj in range(D_MODEL // 8):
                    pltpu.make_async_copy(
                        o_v.at[pl.ds(8 * tj, 8)],
                        out_hbm.at[s, pl.ds(8 * tj, 8), pl.ds(tb * TILE_W, TILE_W)],
                        osem,
                    ).wait()
            return 0

        lax.fori_loop(0, per_w // 2, pair_body, 0)

    return lookup


def kernel(x, lut):
    b, s = x.shape
    lutt = lut.T
    n_full = (VOCAB // TILE_W) * TILE_W
    tail = jnp.pad(lutt[:, n_full:], ((0, 0), (0, TILE_W - (VOCAB - n_full))))
    lut2 = _pack_kernel()(lutt, tail)
    xt = x.T.astype(jnp.int32)
    outk = _lookup_kernel(s, b)(xt, lut2)
    return jnp.transpose(outk, (2, 0, 1))


# speedup vs baseline: 1.0110x; 1.0110x over previous
import functools
import math

import jax
import jax.numpy as jnp
from jax import lax
from jax.experimental import pallas as pl
from jax.experimental.pallas import tpu as pltpu
from jax.experimental.pallas import tpu_sc as plsc

D_MODEL = 64
SCALE = math.sqrt(D_MODEL)
LANES = 16
VOCAB = 1000000
TILE_W = 128
PK = 4
SB_W = PK * TILE_W
LK = 2
LU_W = LK * TILE_W


def _pack_kernel():
    info = plsc.get_sparse_core_info()
    nw = info.num_cores * info.num_subcores
    n_super = VOCAB // SB_W
    rem = VOCAB - n_super * SB_W
    per_w = -(-n_super // nw)
    assert per_w % 2 == 0

    mesh = plsc.VectorSubcoreMesh(core_axis_name="c", subcore_axis_name="s")

    @functools.partial(
        pl.kernel,
        mesh=mesh,
        out_type=jax.ShapeDtypeStruct((VOCAB // 2, 2 * D_MODEL), jnp.float32),
        scratch_types=[
            pltpu.VMEM((D_MODEL, SB_W), jnp.float32),
            pltpu.VMEM((D_MODEL, SB_W), jnp.float32),
            pltpu.VMEM((SB_W // 2, 2 * D_MODEL), jnp.float32),
            pltpu.VMEM((D_MODEL, TILE_W), jnp.float32),
            pltpu.SemaphoreType.DMA,
            pltpu.SemaphoreType.DMA,
        ],
        compiler_params=pltpu.CompilerParams(needs_layout_passes=False),
    )
    def pack(lutt_hbm, tail_hbm, out_hbm, b0_v, b1_v, s_v, t_v, sem0, sem1):
        wid = lax.axis_index("s") * info.num_cores + lax.axis_index("c")
        start = wid * per_w
        bufs = (b0_v, b1_v)
        sems = (sem0, sem1)
        iota = lax.iota(jnp.int32, LANES)

        def fire(sid, buf):
            @pl.when(sid < n_super)
            def _():
                pltpu.async_copy(
                    lutt_hbm.at[:, pl.ds(sid * SB_W, SB_W)],
                    bufs[buf],
                    sems[buf],
                )

        def drain(sid, buf):
            @pl.when(sid < n_super)
            def _():
                pltpu.make_async_copy(
                    lutt_hbm.at[:, pl.ds(sid * SB_W, SB_W)],
                    bufs[buf],
                    sems[buf],
                ).wait()

        fire(start, 0)

        def pair_body(g, _):
            for b in range(2):
                k = 2 * g + b
                sid = start + k
                p, q = b, 1 - b
                drain(sid, p)

                @pl.when(k + 1 < per_w)
                def _():
                    fire(sid + 1, q)

                @pl.when(sid < n_super)
                def _():
                    @plsc.parallel_loop(0, SB_W // 2, step=1, unroll=2)
                    def _(r):
                        base = ((r >> 6) << 7) + ((r & 63) << 1)
                        for h in range(2):
                            col = jnp.full((LANES,), base + h, jnp.int32)
                            for jg in range(D_MODEL // LANES):
                                v = plsc.load_gather(
                                    bufs[p], [jg * LANES + iota, col]
                                )
                                s_v[r, pl.ds(h * D_MODEL + jg * LANES, LANES)] = v

                    pltpu.sync_copy(
                        s_v, out_hbm.at[pl.ds(sid * (SB_W // 2), SB_W // 2)]
                    )
            return 0

        lax.fori_loop(0, per_w // 2, pair_body, 0)

        @pl.when(wid == nw - 1)
        def _():
            pltpu.async_copy(tail_hbm, t_v, sem0)
            pltpu.make_async_copy(tail_hbm, t_v, sem0).wait()

            @plsc.parallel_loop(0, rem // 2, step=1, unroll=2)
            def _(r):
                for h in range(2):
                    col = jnp.full((LANES,), 2 * r + h, jnp.int32)
                    for jg in range(D_MODEL // LANES):
                        v = plsc.load_gather(t_v, [jg * LANES + iota, col])
                        s_v[r, pl.ds(h * D_MODEL + jg * LANES, LANES)] = v

            pltpu.sync_copy(
                s_v.at[pl.ds(0, rem // 2)],
                out_hbm.at[pl.ds((VOCAB - rem) // 2, rem // 2)],
            )

    return pack


def _lookup_kernel(n_s, n_b):
    info = plsc.get_sparse_core_info()
    nw = info.num_cores * info.num_subcores
    n_sup = n_s * n_b // LU_W
    per_w = n_sup // nw
    assert n_sup % nw == 0 and per_w % 2 == 0
    sup_per_s = n_b // LU_W
    s_shift = sup_per_s.bit_length() - 1
    assert 1 << s_shift == sup_per_s

    mesh = plsc.VectorSubcoreMesh(core_axis_name="c", subcore_axis_name="s")

    @functools.partial(
        pl.kernel,
        mesh=mesh,
        out_type=jax.ShapeDtypeStruct((n_s, D_MODEL, n_b), jnp.float32),
        scratch_types=[
            pltpu.VMEM((LU_W,), jnp.int32),
            pltpu.VMEM((LU_W,), jnp.int32),
            pltpu.VMEM((LK, TILE_W), jnp.int32),
            pltpu.VMEM((LK, TILE_W), jnp.int32),
            pltpu.VMEM((LU_W, 2 * D_MODEL), jnp.float32),
            pltpu.VMEM((LU_W, 2 * D_MODEL), jnp.float32),
            pltpu.VMEM((D_MODEL, LU_W), jnp.float32),
            pltpu.SemaphoreType.DMA,
            pltpu.SemaphoreType.DMA,
        ],
        compiler_params=pltpu.CompilerParams(needs_layout_passes=False),
    )
    def lookup(xt_hbm, lut2_hbm, out_hbm, i0_v, i1_v, g0_v, g1_v,
               r0_v, r1_v, o_v, sem0, sem1):
        wid = lax.axis_index("s") * info.num_cores + lax.axis_index("c")
        start = wid * per_w
        idxs = (i0_v, i1_v)
        gidxs = (g0_v, g1_v)
        rows = (r0_v, r1_v)
        sems = (sem0, sem1)
        iota = lax.iota(jnp.int32, LANES)

        def fire(u, buf):
            s = u >> s_shift
            t = u & (sup_per_s - 1)
            pltpu.sync_copy(xt_hbm.at[s, pl.ds(t * LU_W, LU_W)], idxs[buf])
            for j in range(LK):
                for k in range(TILE_W // LANES):
                    ids = idxs[buf][pl.ds(j * TILE_W + k * LANES, LANES)]
                    gidxs[buf][j, pl.ds(k * LANES, LANES)] = ids >> 1
            for j in range(LK):
                pltpu.async_copy(
                    lut2_hbm.at[gidxs[buf].at[j]],
                    rows[buf].at[pl.ds(j * TILE_W, TILE_W)],
                    sems[buf],
                )

        def drain(buf):
            for j in range(LK):
                pltpu.make_async_copy(
                    lut2_hbm.at[gidxs[buf].at[j]],
                    rows[buf].at[pl.ds(j * TILE_W, TILE_W)],
                    sems[buf],
                ).wait()

        fire(start, 0)

        def pair_body(g, _):
            for b in range(2):
                k = 2 * g + b
                u = start + k
                p, q = b, 1 - b
                drain(p)

                @pl.when(k + 1 < per_w)
                def _():
                    fire(u + 1, q)

                for bg in range(LU_W // LANES):
                    ids = idxs[p][pl.ds(bg * LANES, LANES)]
                    half = (ids & 1) << 6
                    brow = bg * LANES + iota

                    @plsc.parallel_loop(0, D_MODEL, step=1, unroll=4)
                    def _(j):
                        v = plsc.load_gather(rows[p], [brow, half + j])
                        o_v[j, pl.ds(bg * LANES, LANES)] = v * SCALE

                s = u >> s_shift
                t = u & (sup_per_s - 1)
                pltpu.sync_copy(o_v, out_hbm.at[s, :, pl.ds(t * LU_W, LU_W)])
            return 0

        lax.fori_loop(0, per_w // 2, pair_body, 0)

    return lookup


def kernel(x, lut):
    b, s = x.shape
    lutt = lut.T
    n_full = (VOCAB // SB_W) * SB_W
    tail = jnp.pad(lutt[:, n_full:], ((0, 0), (0, TILE_W - (VOCAB - n_full))))
    lut2 = _pack_kernel()(lutt, tail)
    xt = x.T.astype(jnp.int32)
    outk = _lookup_kernel(s, b)(xt, lut2)
    return jnp.transpose(outk, (2, 0, 1))


# speedup vs baseline: 1.0462x; 1.0348x over previous
import functools
import math

import jax
import jax.numpy as jnp
from jax import lax
from jax.experimental import pallas as pl
from jax.experimental.pallas import tpu as pltpu
from jax.experimental.pallas import tpu_sc as plsc

D_MODEL = 64
SCALE = math.sqrt(D_MODEL)
LANES = 16
VOCAB = 1000000
TILE_W = 128
SB_W = 2 * TILE_W
LU_W = 2 * TILE_W


def _pack_kernel():
    info = plsc.get_sparse_core_info()
    nw = info.num_cores * info.num_subcores
    n_super = VOCAB // SB_W
    rem = VOCAB - n_super * SB_W
    per_w = -(-n_super // nw)
    if per_w % 2:
        per_w += 1

    mesh = plsc.VectorSubcoreMesh(core_axis_name="c", subcore_axis_name="s")

    @functools.partial(
        pl.kernel,
        mesh=mesh,
        out_type=jax.ShapeDtypeStruct((VOCAB // 2, 2 * D_MODEL), jnp.float32),
        scratch_types=[
            pltpu.VMEM((D_MODEL, SB_W), jnp.float32),
            pltpu.VMEM((D_MODEL, SB_W), jnp.float32),
            pltpu.VMEM((SB_W // 2, 2 * D_MODEL), jnp.float32),
            pltpu.VMEM((SB_W // 2, 2 * D_MODEL), jnp.float32),
            pltpu.VMEM((D_MODEL, TILE_W), jnp.float32),
            pltpu.SemaphoreType.DMA,
            pltpu.SemaphoreType.DMA,
            pltpu.SemaphoreType.DMA,
            pltpu.SemaphoreType.DMA,
        ],
        compiler_params=pltpu.CompilerParams(needs_layout_passes=False),
    )
    def pack(lutt_hbm, tail_hbm, out_hbm, b0_v, b1_v, s0_v, s1_v, t_v,
             isem0, isem1, osem0, osem1):
        wid = lax.axis_index("s") * info.num_cores + lax.axis_index("c")
        start = wid * per_w
        bufs = (b0_v, b1_v)
        stgs = (s0_v, s1_v)
        isems = (isem0, isem1)
        osems = (osem0, osem1)
        iota = lax.iota(jnp.int32, LANES)

        def fire_in(sid, buf):
            @pl.when(sid < n_super)
            def _():
                pltpu.async_copy(
                    lutt_hbm.at[:, pl.ds(sid * SB_W, SB_W)], bufs[buf],
                    isems[buf],
                )

        def drain_in(sid, buf):
            @pl.when(sid < n_super)
            def _():
                pltpu.make_async_copy(
                    lutt_hbm.at[:, pl.ds(sid * SB_W, SB_W)], bufs[buf],
                    isems[buf],
                ).wait()

        def fire_out(sid, buf):
            @pl.when(sid < n_super)
            def _():
                pltpu.async_copy(
                    stgs[buf],
                    out_hbm.at[pl.ds(sid * (SB_W // 2), SB_W // 2)],
                    osems[buf],
                )

        def drain_out(sid, buf):
            @pl.when(sid < n_super)
            def _():
                pltpu.make_async_copy(
                    stgs[buf],
                    out_hbm.at[pl.ds(sid * (SB_W // 2), SB_W // 2)],
                    osems[buf],
                ).wait()

        fire_in(start, 0)

        def pair_body(g, _):
            for b in range(2):
                k = 2 * g + b
                sid = start + k
                p, q = b, 1 - b
                drain_in(sid, p)

                @pl.when(k + 1 < per_w)
                def _():
                    fire_in(sid + 1, q)

                @pl.when(k >= 1)
                def _():
                    drain_out(sid - 1, q)

                @pl.when(sid < n_super)
                def _():
                    @plsc.parallel_loop(0, SB_W // 2, step=1, unroll=2)
                    def _(r):
                        base = ((r >> 6) << 7) + ((r & 63) << 1)
                        for h in range(2):
                            col = jnp.full((LANES,), base + h, jnp.int32)
                            for jg in range(D_MODEL // LANES):
                                v = plsc.load_gather(
                                    bufs[p], [jg * LANES + iota, col]
                                )
                                stgs[p][r, pl.ds(h * D_MODEL + jg * LANES,
                                                 LANES)] = v

                fire_out(sid, p)
            return 0

        lax.fori_loop(0, per_w // 2, pair_body, 0)
        drain_out(start + per_w - 1, (per_w - 1) & 1)

        @pl.when(wid == nw - 1)
        def _():
            pltpu.async_copy(tail_hbm, t_v, isem0)
            pltpu.make_async_copy(tail_hbm, t_v, isem0).wait()

            @plsc.parallel_loop(0, rem // 2, step=1, unroll=2)
            def _(r):
                for h in range(2):
                    col = jnp.full((LANES,), 2 * r + h, jnp.int32)
                    for jg in range(D_MODEL // LANES):
                        v = plsc.load_gather(t_v, [jg * LANES + iota, col])
                        s0_v[r, pl.ds(h * D_MODEL + jg * LANES, LANES)] = v

            pltpu.sync_copy(
                s0_v.at[pl.ds(0, rem // 2)],
                out_hbm.at[pl.ds((VOCAB - rem) // 2, rem // 2)],
            )

    return pack


def _lookup_kernel(n_s, n_b):
    info = plsc.get_sparse_core_info()
    nw = info.num_cores * info.num_subcores
    n_sup = n_s * n_b // LU_W
    per_w = n_sup // nw
    assert n_sup % nw == 0 and per_w % 2 == 0
    sup_per_s = n_b // LU_W
    s_shift = sup_per_s.bit_length() - 1
    assert 1 << s_shift == sup_per_s

    mesh = plsc.VectorSubcoreMesh(core_axis_name="c", subcore_axis_name="s")

    @functools.partial(
        pl.kernel,
        mesh=mesh,
        out_type=jax.ShapeDtypeStruct((n_s, D_MODEL, n_b), jnp.float32),
        scratch_types=[
            pltpu.VMEM((LU_W,), jnp.int32),
            pltpu.VMEM((LU_W,), jnp.int32),
            pltpu.VMEM((2, TILE_W), jnp.int32),
            pltpu.VMEM((2, TILE_W), jnp.int32),
            pltpu.VMEM((LU_W, 2 * D_MODEL), jnp.float32),
            pltpu.VMEM((LU_W, 2 * D_MODEL), jnp.float32),
            pltpu.VMEM((D_MODEL, LU_W), jnp.float32),
            pltpu.VMEM((D_MODEL, LU_W), jnp.float32),
            pltpu.SemaphoreType.DMA,
            pltpu.SemaphoreType.DMA,
            pltpu.SemaphoreType.DMA,
            pltpu.SemaphoreType.DMA,
            pltpu.SemaphoreType.DMA,
            pltpu.SemaphoreType.DMA,
        ],
        compiler_params=pltpu.CompilerParams(needs_layout_passes=False),
    )
    def lookup(xt_hbm, lut2_hbm, out_hbm, i0_v, i1_v, g0_v, g1_v,
               r0_v, r1_v, o0_v, o1_v,
               isem0, isem1, rsem0, rsem1, osem0, osem1):
        wid = lax.axis_index("s") * info.num_cores + lax.axis_index("c")
        start = wid * per_w
        idxs = (i0_v, i1_v)
        gidxs = (g0_v, g1_v)
        rows = (r0_v, r1_v)
        outs = (o0_v, o1_v)
        isems = (isem0, isem1)
        rsems = (rsem0, rsem1)
        osems = (osem0, osem1)
        iota = lax.iota(jnp.int32, LANES)

        def uview(u):
            return u >> s_shift, u & (sup_per_s - 1)

        def fire_idx(u, buf):
            s, t = uview(u)
            pltpu.async_copy(
                xt_hbm.at[s, pl.ds(t * LU_W, LU_W)], idxs[buf], isems[buf]
            )

        def wait_idx(u, buf):
            s, t = uview(u)
            pltpu.make_async_copy(
                xt_hbm.at[s, pl.ds(t * LU_W, LU_W)], idxs[buf], isems[buf]
            ).wait()

        def fire_rows(buf):
            for j in range(2):
                for k in range(TILE_W // LANES):
                    ids = idxs[buf][pl.ds(j * TILE_W + k * LANES, LANES)]
                    gidxs[buf][j, pl.ds(k * LANES, LANES)] = ids >> 1
            for j in range(2):
                pltpu.async_copy(
                    lut2_hbm.at[gidxs[buf].at[j]],
                    rows[buf].at[pl.ds(j * TILE_W, TILE_W)],
                    rsems[buf],
                )

        def drain_rows(buf):
            for j in range(2):
                pltpu.make_async_copy(
                    lut2_hbm.at[gidxs[buf].at[j]],
                    rows[buf].at[pl.ds(j * TILE_W, TILE_W)],
                    rsems[buf],
                ).wait()

        def fire_out(u, buf):
            s, t = uview(u)
            pltpu.async_copy(
                outs[buf], out_hbm.at[s, :, pl.ds(t * LU_W, LU_W)], osems[buf]
            )

        def drain_out(u, buf):
            s, t = uview(u)
            pltpu.make_async_copy(
                outs[buf], out_hbm.at[s, :, pl.ds(t * LU_W, LU_W)], osems[buf]
            ).wait()

        fire_idx(start, 0)
        fire_idx(start + 1, 1)
        wait_idx(start, 0)
        fire_rows(0)

        def pair_body(g, _):
            for b in range(2):
                k = 2 * g + b
                u = start + k
                p, q = b, 1 - b
                drain_rows(p)

                @pl.when(k + 1 < per_w)
                def _():
                    wait_idx(u + 1, q)
                    fire_rows(q)

                for bg in range(LU_W // LANES):
                    ids = idxs[p][pl.ds(bg * LANES, LANES)]
                    half = (ids & 1) << 6
                    brow = bg * LANES + iota

                    @plsc.parallel_loop(0, D_MODEL, step=1, unroll=4)
                    def _(j):
                        v = plsc.load_gather(rows[p], [brow, half + j])
                        outs[p][j, pl.ds(bg * LANES, LANES)] = v * SCALE

                @pl.when(k + 2 < per_w)
                def _():
                    fire_idx(u + 2, p)

                @pl.when(k >= 1)
                def _():
                    drain_out(u - 1, q)

                fire_out(u, p)
            return 0

        lax.fori_loop(0, per_w // 2, pair_body, 0)
        drain_out(start + per_w - 1, (per_w - 1) & 1)

    return lookup


def kernel(x, lut):
    b, s = x.shape
    lutt = lut.T
    n_full = (VOCAB // SB_W) * SB_W
    tail = jnp.pad(lutt[:, n_full:], ((0, 0), (0, TILE_W - (VOCAB - n_full))))
    lut2 = _pack_kernel()(lutt, tail)
    xt = x.T.astype(jnp.int32)
    outk = _lookup_kernel(s, b)(xt, lut2)
    return jnp.transpose(outk, (2, 0, 1))


# speedup vs baseline: 1.0847x; 1.0368x over previous
import functools
import math

import jax
import jax.numpy as jnp
from jax import lax
from jax.experimental import pallas as pl
from jax.experimental.pallas import tpu as pltpu
from jax.experimental.pallas import tpu_sc as plsc

D_MODEL = 64
SCALE = math.sqrt(D_MODEL)
LANES = 16
VOCAB = 1000000
TILE_W = 128
SB_W = 2 * TILE_W
LU_W = 2 * TILE_W


def _pack_kernel():
    info = plsc.get_sparse_core_info()
    nw = info.num_cores * info.num_subcores
    n_super = VOCAB // SB_W
    rem = VOCAB - n_super * SB_W
    per_w = -(-n_super // nw)
    if per_w % 2:
        per_w += 1

    mesh = plsc.VectorSubcoreMesh(core_axis_name="c", subcore_axis_name="s")

    @functools.partial(
        pl.kernel,
        mesh=mesh,
        out_type=jax.ShapeDtypeStruct((VOCAB // 2, 2 * D_MODEL), jnp.float32),
        scratch_types=[
            pltpu.VMEM((D_MODEL, SB_W), jnp.float32),
            pltpu.VMEM((D_MODEL, SB_W), jnp.float32),
            pltpu.VMEM((SB_W // 2, 2 * D_MODEL), jnp.float32),
            pltpu.VMEM((SB_W // 2, 2 * D_MODEL), jnp.float32),
            pltpu.VMEM((D_MODEL, TILE_W), jnp.float32),
            pltpu.SemaphoreType.DMA,
            pltpu.SemaphoreType.DMA,
            pltpu.SemaphoreType.DMA,
            pltpu.SemaphoreType.DMA,
        ],
        compiler_params=pltpu.CompilerParams(needs_layout_passes=False),
    )
    def pack(lutt_hbm, tail_hbm, out_hbm, b0_v, b1_v, s0_v, s1_v, t_v,
             isem0, isem1, osem0, osem1):
        wid = lax.axis_index("s") * info.num_cores + lax.axis_index("c")
        start = wid * per_w
        bufs = (b0_v, b1_v)
        stgs = (s0_v, s1_v)
        isems = (isem0, isem1)
        osems = (osem0, osem1)
        iota = lax.iota(jnp.int32, LANES)

        def fire_in(sid, buf):
            @pl.when(sid < n_super)
            def _():
                pltpu.async_copy(
                    lutt_hbm.at[:, pl.ds(sid * SB_W, SB_W)], bufs[buf],
                    isems[buf],
                )

        def drain_in(sid, buf):
            @pl.when(sid < n_super)
            def _():
                pltpu.make_async_copy(
                    lutt_hbm.at[:, pl.ds(sid * SB_W, SB_W)], bufs[buf],
                    isems[buf],
                ).wait()

        def fire_out(sid, buf):
            @pl.when(sid < n_super)
            def _():
                pltpu.async_copy(
                    stgs[buf],
                    out_hbm.at[pl.ds(sid * (SB_W // 2), SB_W // 2)],
                    osems[buf],
                )

        def drain_out(sid, buf):
            @pl.when(sid < n_super)
            def _():
                pltpu.make_async_copy(
                    stgs[buf],
                    out_hbm.at[pl.ds(sid * (SB_W // 2), SB_W // 2)],
                    osems[buf],
                ).wait()

        fire_in(start, 0)

        def pair_body(g, _):
            for b in range(2):
                k = 2 * g + b
                sid = start + k
                p, q = b, 1 - b
                drain_in(sid, p)

                @pl.when(k + 1 < per_w)
                def _():
                    fire_in(sid + 1, q)

                @pl.when(k >= 1)
                def _():
                    drain_out(sid - 1, q)

                @pl.when(sid < n_super)
                def _():
                    @plsc.parallel_loop(0, SB_W // 2, step=1, unroll=8)
                    def _(r):
                        base = ((r >> 6) << 7) + ((r & 63) << 1)
                        for h in range(2):
                            col = jnp.full((LANES,), base + h, jnp.int32)
                            for jg in range(D_MODEL // LANES):
                                v = plsc.load_gather(
                                    bufs[p], [jg * LANES + iota, col]
                                )
                                stgs[p][r, pl.ds(h * D_MODEL + jg * LANES,
                                                 LANES)] = v

                fire_out(sid, p)
            return 0

        lax.fori_loop(0, per_w // 2, pair_body, 0)
        drain_out(start + per_w - 1, (per_w - 1) & 1)

        @pl.when(wid == nw - 1)
        def _():
            pltpu.async_copy(tail_hbm, t_v, isem0)
            pltpu.make_async_copy(tail_hbm, t_v, isem0).wait()

            @plsc.parallel_loop(0, rem // 2, step=1, unroll=2)
            def _(r):
                for h in range(2):
                    col = jnp.full((LANES,), 2 * r + h, jnp.int32)
                    for jg in range(D_MODEL // LANES):
                        v = plsc.load_gather(t_v, [jg * LANES + iota, col])
                        s0_v[r, pl.ds(h * D_MODEL + jg * LANES, LANES)] = v

            pltpu.sync_copy(
                s0_v.at[pl.ds(0, rem // 2)],
                out_hbm.at[pl.ds((VOCAB - rem) // 2, rem // 2)],
            )

    return pack


def _lookup_kernel(n_s, n_b):
    info = plsc.get_sparse_core_info()
    nw = info.num_cores * info.num_subcores
    n_sup = n_s * n_b // LU_W
    per_w = n_sup // nw
    assert n_sup % nw == 0 and per_w % 2 == 0
    sup_per_s = n_b // LU_W
    s_shift = sup_per_s.bit_length() - 1
    assert 1 << s_shift == sup_per_s

    mesh = plsc.VectorSubcoreMesh(core_axis_name="c", subcore_axis_name="s")

    @functools.partial(
        pl.kernel,
        mesh=mesh,
        out_type=jax.ShapeDtypeStruct((n_s, D_MODEL, n_b), jnp.float32),
        scratch_types=[
            pltpu.VMEM((LU_W,), jnp.int32),
            pltpu.VMEM((LU_W,), jnp.int32),
            pltpu.VMEM((2, TILE_W), jnp.int32),
            pltpu.VMEM((2, TILE_W), jnp.int32),
            pltpu.VMEM((LU_W, 2 * D_MODEL), jnp.float32),
            pltpu.VMEM((LU_W, 2 * D_MODEL), jnp.float32),
            pltpu.VMEM((D_MODEL, LU_W), jnp.float32),
            pltpu.VMEM((D_MODEL, LU_W), jnp.float32),
            pltpu.SemaphoreType.DMA,
            pltpu.SemaphoreType.DMA,
            pltpu.SemaphoreType.DMA,
            pltpu.SemaphoreType.DMA,
            pltpu.SemaphoreType.DMA,
            pltpu.SemaphoreType.DMA,
        ],
        compiler_params=pltpu.CompilerParams(needs_layout_passes=False),
    )
    def lookup(xt_hbm, lut2_hbm, out_hbm, i0_v, i1_v, g0_v, g1_v,
               r0_v, r1_v, o0_v, o1_v,
               isem0, isem1, rsem0, rsem1, osem0, osem1):
        wid = lax.axis_index("s") * info.num_cores + lax.axis_index("c")
        start = wid * per_w
        idxs = (i0_v, i1_v)
        gidxs = (g0_v, g1_v)
        rows = (r0_v, r1_v)
        outs = (o0_v, o1_v)
        isems = (isem0, isem1)
        rsems = (rsem0, rsem1)
        osems = (osem0, osem1)
        iota = lax.iota(jnp.int32, LANES)

        def uview(u):
            return u >> s_shift, u & (sup_per_s - 1)

        def fire_idx(u, buf):
            s, t = uview(u)
            pltpu.async_copy(
                xt_hbm.at[s, pl.ds(t * LU_W, LU_W)], idxs[buf], isems[buf]
            )

        def wait_idx(u, buf):
            s, t = uview(u)
            pltpu.make_async_copy(
                xt_hbm.at[s, pl.ds(t * LU_W, LU_W)], idxs[buf], isems[buf]
            ).wait()

        def fire_rows(buf):
            for j in range(2):
                for k in range(TILE_W // LANES):
                    ids = idxs[buf][pl.ds(j * TILE_W + k * LANES, LANES)]
                    gidxs[buf][j, pl.ds(k * LANES, LANES)] = ids >> 1
            for j in range(2):
                pltpu.async_copy(
                    lut2_hbm.at[gidxs[buf].at[j]],
                    rows[buf].at[pl.ds(j * TILE_W, TILE_W)],
                    rsems[buf],
                )

        def drain_rows(buf):
            for j in range(2):
                pltpu.make_async_copy(
                    lut2_hbm.at[gidxs[buf].at[j]],
                    rows[buf].at[pl.ds(j * TILE_W, TILE_W)],
                    rsems[buf],
                ).wait()

        def fire_out(u, buf):
            s, t = uview(u)
            pltpu.async_copy(
                outs[buf], out_hbm.at[s, :, pl.ds(t * LU_W, LU_W)], osems[buf]
            )

        def drain_out(u, buf):
            s, t = uview(u)
            pltpu.make_async_copy(
                outs[buf], out_hbm.at[s, :, pl.ds(t * LU_W, LU_W)], osems[buf]
            ).wait()

        fire_idx(start, 0)
        fire_idx(start + 1, 1)
        wait_idx(start, 0)
        fire_rows(0)

        def pair_body(g, _):
            for b in range(2):
                k = 2 * g + b
                u = start + k
                p, q = b, 1 - b
                drain_rows(p)

                @pl.when(k + 1 < per_w)
                def _():
                    wait_idx(u + 1, q)
                    fire_rows(q)

                for bg in range(LU_W // LANES):
                    ids = idxs[p][pl.ds(bg * LANES, LANES)]
                    half = (ids & 1) << 6
                    brow = bg * LANES + iota

                    @plsc.parallel_loop(0, D_MODEL, step=1, unroll=16)
                    def _(j):
                        v = plsc.load_gather(rows[p], [brow, half + j])
                        outs[p][j, pl.ds(bg * LANES, LANES)] = v * SCALE

                @pl.when(k + 2 < per_w)
                def _():
                    fire_idx(u + 2, p)

                @pl.when(k >= 1)
                def _():
                    drain_out(u - 1, q)

                fire_out(u, p)
            return 0

        lax.fori_loop(0, per_w // 2, pair_body, 0)
        drain_out(start + per_w - 1, (per_w - 1) & 1)

    return lookup


def kernel(x, lut):
    b, s = x.shape
    lutt = lut.T
    n_full = (VOCAB // SB_W) * SB_W
    tail = jnp.pad(lutt[:, n_full:], ((0, 0), (0, TILE_W - (VOCAB - n_full))))
    lut2 = _pack_kernel()(lutt, tail)
    xt = x.T.astype(jnp.int32)
    outk = _lookup_kernel(s, b)(xt, lut2)
    return jnp.transpose(outk, (2, 0, 1))


# speedup vs baseline: 1.5779x; 1.4547x over previous
import functools
import math

import jax
import jax.numpy as jnp
from jax import lax
from jax.experimental import pallas as pl
from jax.experimental.pallas import tpu as pltpu
from jax.experimental.pallas import tpu_sc as plsc

D_MODEL = 64
SCALE = math.sqrt(D_MODEL)
LANES = 16
IDX_ROW = 128
ROWS_PER_CHUNK = 2
CHUNK = IDX_ROW * ROWS_PER_CHUNK


def _make_kernel(n_total):
    info = plsc.get_sparse_core_info()
    nw = info.num_cores * info.num_subcores
    per_w = n_total // nw
    n_chunks = per_w // CHUNK
    assert per_w % CHUNK == 0 and n_chunks % 2 == 0

    mesh = plsc.VectorSubcoreMesh(core_axis_name="c", subcore_axis_name="s")

    @functools.partial(
        pl.kernel,
        mesh=mesh,
        out_type=jax.ShapeDtypeStruct((n_total, D_MODEL), jnp.float32),
        scratch_types=[
            pltpu.VMEM((2, ROWS_PER_CHUNK, IDX_ROW), jnp.int32),
            pltpu.VMEM((2, ROWS_PER_CHUNK, IDX_ROW), jnp.int32),
            pltpu.VMEM((CHUNK, 2 * D_MODEL), jnp.float32),
            pltpu.VMEM((CHUNK, 2 * D_MODEL), jnp.float32),
            pltpu.VMEM((CHUNK, D_MODEL), jnp.float32),
            pltpu.SemaphoreType.DMA,
            pltpu.SemaphoreType.DMA,
        ],
        compiler_params=pltpu.CompilerParams(needs_layout_passes=False),
    )
    def emb(x_hbm, lut2_hbm, out_hbm, idx_v, gidx_v, g0_v, g1_v, stg_v,
            sem0, sem1):
        wid = lax.axis_index("s") * info.num_cores + lax.axis_index("c")
        idx_row_base = wid * (per_w // IDX_ROW)
        out_base = wid * per_w
        gath = (g0_v, g1_v)
        sems = (sem0, sem1)
        iota = lax.iota(jnp.int32, LANES)

        def start_gathers(g, buf):
            pltpu.sync_copy(
                x_hbm.at[pl.ds(idx_row_base + g * ROWS_PER_CHUNK, ROWS_PER_CHUNK)],
                idx_v.at[buf],
            )
            for j in range(ROWS_PER_CHUNK):
                for k in range(IDX_ROW // LANES):
                    ids = idx_v[buf, j, pl.ds(k * LANES, LANES)]
                    gidx_v[buf, j, pl.ds(k * LANES, LANES)] = ids >> 1
            for j in range(ROWS_PER_CHUNK):
                pltpu.async_copy(
                    lut2_hbm.at[gidx_v.at[buf, j]],
                    gath[buf].at[pl.ds(j * IDX_ROW, IDX_ROW)],
                    sems[buf],
                )

        def drain_gathers(buf):
            for j in range(ROWS_PER_CHUNK):
                pltpu.make_async_copy(
                    lut2_hbm.at[gidx_v.at[buf, j]],
                    gath[buf].at[pl.ds(j * IDX_ROW, IDX_ROW)],
                    sems[buf],
                ).wait()

        start_gathers(0, 0)

        def pair_body(g2, _):
            for b in range(2):
                g = 2 * g2 + b
                p, q = b, 1 - b
                drain_gathers(p)

                @pl.when(g + 1 < n_chunks)
                def _():
                    start_gathers(g + 1, q)

                @plsc.parallel_loop(0, CHUNK, step=1, unroll=4)
                def _(i):
                    row = jnp.full((LANES,), i, jnp.int32)
                    ids = plsc.load_gather(
                        idx_v, [jnp.full((LANES,), p, jnp.int32),
                                row >> 7, jnp.full((LANES,), i & 127, jnp.int32)]
                    )
                    half = (ids & 1) << 6
                    for j in range(D_MODEL // LANES):
                        v = plsc.load_gather(gath[p], [row, half + (j * LANES) + iota])
                        stg_v[i, pl.ds(j * LANES, LANES)] = v * SCALE

                pltpu.sync_copy(
                    stg_v, out_hbm.at[pl.ds(out_base + g * CHUNK, CHUNK)]
                )
            return 0

        lax.fori_loop(0, n_chunks // 2, pair_body, 0)

    return emb


def kernel(x, lut):
    b, s = x.shape
    n = b * s
    xi = x.reshape(n // IDX_ROW, IDX_ROW).astype(jnp.int32)
    lut2 = lut.reshape(lut.shape[0] // 2, 2 * D_MODEL)
    out = _make_kernel(n)(xi, lut2)
    return out.reshape(b, s, D_MODEL)
